# baseline (speedup 1.0000x reference)
import functools

import jax
import jax.numpy as jnp
from jax import lax
from jax.experimental import pallas as pl
from jax.experimental.pallas import tpu as pltpu
from jax.experimental.pallas import tpu_sc as plsc

ENT_TOT = 1000000
REL_TOT = 1000
B = 16384
DIM = 64
PDIM = 128
NC = 2
NS = 16
NW = NC * NS
ROWS_PER_W = B // NW
CHUNK = 128
NCHUNK = ROWS_PER_W // CHUNK
GROUPS = CHUNK // 16

ENT_NBLK = (ENT_TOT + PDIM - 1) // PDIM
ENT_GPT = (ENT_NBLK + NW - 1) // NW
REL_NBLK = (REL_TOT + PDIM - 1) // PDIM
IBW = 133

_F32 = jnp.float32
_MAGIC = 0x5F3759DF


def _rsqrt(x):
    i = plsc.bitcast(x, jnp.int32)
    y = plsc.bitcast(jnp.int32(_MAGIC) - (i >> 1), _F32)
    for _ in range(3):
        y = y * (_F32(1.5) - _F32(0.5) * x * y * y)
    return y


def _sqrt(x):
    return x * _rsqrt(jnp.maximum(x, _F32(1e-30)))


def _pack_body(ent_t, rel_t, ent2, rel2, ib0, ib1, ob, sem0, sem1):
    cid = lax.axis_index("c")
    sid = lax.axis_index("s")
    wid = sid * NC + cid
    lane = lax.broadcasted_iota(jnp.int32, (16,), 0)
    jvs = [k * 16 + lane for k in range(4)]

    def colstart(bid, tot):
        del tot
        return pl.multiple_of(bid * PDIM, PDIM)

    def transpose_block(ib):
        def rows(r4, carry):
            for u in range(4):
                r = r4 * 4 + u
                ca = jnp.zeros((16,), jnp.int32) + 2 * r
                cb = ca + 1
                for k in range(4):
                    ob[r, pl.ds(k * 16, 16)] = plsc.load_gather(
                        ib, [jvs[k], ca])
                    ob[r, pl.ds(64 + k * 16, 16)] = plsc.load_gather(
                        ib, [jvs[k], cb])
            return carry

        lax.fori_loop(0, 16, rows, 0)

    def ent_in_cp(g, ib, sem):
        c0 = colstart(g * NW + wid, ENT_TOT)
        return pltpu.make_async_copy(
            ent_t.at[:, pl.ds(c0, PDIM)], ib.at[:, pl.ds(0, PDIM)], sem)

    ent_in_cp(0, ib0, sem0).start()

    def pack2(gg, carry):
        for s, ibs, sems in ((0, (ib0, ib1), (sem0, sem1)),
                             (1, (ib1, ib0), (sem1, sem0))):
            g = 2 * gg + s
            ib, ibn = ibs
            sem, semn = sems
            valid = g * NW + wid < ENT_NBLK - 1
            nvalid = (g + 1) * NW + wid < ENT_NBLK - 1

            @pl.when(valid)
            def _():
                ent_in_cp(g, ib, sem).wait()

            @pl.when(nvalid)
            def _():
                ent_in_cp(g + 1, ibn, semn).start()

            @pl.when(valid)
            def _():
                transpose_block(ib)
                prow = pl.multiple_of((g * NW + wid) * 64, 64)
                pltpu.sync_copy(ob, ent2.at[pl.ds(prow, 64)])
        return carry

    lax.fori_loop(0, (ENT_GPT + 1) // 2, pack2, 0)

    @pl.when(wid == 8)
    def _():
        pltpu.sync_copy(ent_t.at[:, pl.ds(pl.multiple_of(999936, PDIM), PDIM)],
                        ib0.at[:, pl.ds(0, PDIM)])
        transpose_block(ib0)
        pltpu.sync_copy(ob.at[pl.ds(0, 32)], ent2.at[pl.ds(499968, 32)])

    @pl.when(wid < REL_NBLK - 1)
    def _():
        c0 = colstart(wid, REL_TOT)
        pltpu.sync_copy(rel_t.at[:, pl.ds(c0, PDIM)],
                        ib0.at[:, pl.ds(0, PDIM)])
        transpose_block(ib0)
        pltpu.sync_copy(ob, rel2.at[pl.ds(pl.multiple_of(wid * 64, 64), 64)])

    @pl.when(wid == REL_NBLK - 1)
    def _():
        pltpu.sync_copy(rel_t.at[:, pl.ds(pl.multiple_of(896, PDIM), PDIM)],
                        ib0.at[:, pl.ds(0, PDIM)])
        transpose_block(ib0)
        pltpu.sync_copy(ob.at[pl.ds(0, 52)], rel2.at[pl.ds(448, 52)])


@functools.partial(jax.jit, static_argnames=())
def _pack_call(ent_t, rel_t):
    mesh = plsc.VectorSubcoreMesh(core_axis_name="c", subcore_axis_name="s",
                                  num_cores=NC, num_subcores=NS)
    f = pl.kernel(
        _pack_body,
        out_type=(
            jax.ShapeDtypeStruct((ENT_TOT // 2, PDIM), _F32),
            jax.ShapeDtypeStruct((REL_TOT // 2, PDIM), _F32),
        ),
        mesh=mesh,
        compiler_params=pltpu.CompilerParams(needs_layout_passes=False,
                                             disable_bounds_checks=True),
        scratch_types=[
            pltpu.VMEM((DIM, IBW), _F32),
            pltpu.VMEM((DIM, IBW), _F32),
            pltpu.VMEM((DIM, PDIM), _F32),
            pltpu.SemaphoreType.DMA,
            pltpu.SemaphoreType.DMA,
        ],
    )
    return f(ent_t, rel_t)


def _sc_body(head_hbm, rel_hbm, tail_hbm, negv_hbm, ent_hbm, relemb_hbm,
             pos_out, neg_out, dist_out,
             idx_h, idx_r, idx_t, idx_n,
             half_h, half_r, half_t, half_n,
             h_buf, r_buf, t_buf, n_buf,
             inv_h, inv_t, inv_n,
             pos_b, neg1_b, neg2_b, dist_b, sem):
    cid = lax.axis_index("c")
    sid = lax.axis_index("s")
    wid = sid * NC + cid
    base = wid * ROWS_PER_W
    lane = lax.broadcasted_iota(jnp.int32, (16,), 0)
    zero = jnp.zeros((16,), _F32)

    def chunk_body(c, dist_acc):
        cbase = base + c * CHUNK
        pltpu.sync_copy(head_hbm.at[pl.ds(cbase, CHUNK)], idx_h.at[c])
        pltpu.sync_copy(rel_hbm.at[pl.ds(cbase, CHUNK)], idx_r.at[c])
        pltpu.sync_copy(tail_hbm.at[pl.ds(cbase, CHUNK)], idx_t.at[c])
        pltpu.sync_copy(negv_hbm.at[pl.ds(cbase, CHUNK)], idx_n.at[c])

        def halve(g, carry):
            gs = pl.ds(g * 16, 16)
            half_h[gs] = idx_h[c, gs] >> 1
            half_r[gs] = idx_r[c, gs] >> 1
            half_t[gs] = idx_t[c, gs] >> 1
            half_n[gs] = idx_n[c, gs] >> 1
            return carry

        lax.fori_loop(0, GROUPS, halve, 0)

        cp_h = pltpu.async_copy(ent_hbm.at[half_h], h_buf, sem)
        cp_r = pltpu.async_copy(relemb_hbm.at[half_r], r_buf, sem)
        cp_t = pltpu.async_copy(ent_hbm.at[half_t], t_buf, sem)
        cp_n = pltpu.async_copy(ent_hbm.at[half_n], n_buf, sem)
        cp_h.wait()
        cp_r.wait()
        cp_t.wait()
        cp_n.wait()

        def pass1(g, carry):
            gs = pl.ds(g * 16, 16)
            ph = (idx_h[c, gs] & 1) * 64
            pt = (idx_t[c, gs] & 1) * 64
            pn = (idx_n[c, gs] & 1) * 64
            sh_v, st_v, sn_v = zero, zero, zero
            for i in range(16):
                row = g * 16 + i

                def rowsq(buf, pv):
                    bb = pv[i]
                    a = buf[row, pl.ds(bb, 16)]
                    b = buf[row, pl.ds(bb + 16, 16)]
                    cc = buf[row, pl.ds(bb + 32, 16)]
                    d = buf[row, pl.ds(bb + 48, 16)]
                    return jnp.sum(a * a + b * b + cc * cc + d * d)

                sh_v = jnp.where(lane == i, rowsq(h_buf, ph), sh_v)
                st_v = jnp.where(lane == i, rowsq(t_buf, pt), st_v)
                sn_v = jnp.where(lane == i, rowsq(n_buf, pn), sn_v)
            inv_h[gs] = _rsqrt(jnp.maximum(sh_v, _F32(1e-24)))
            inv_t[gs] = _rsqrt(jnp.maximum(st_v, _F32(1e-24)))
            inv_n[gs] = _rsqrt(jnp.maximum(sn_v, _F32(1e-24)))
            return carry

        lax.fori_loop(0, GROUPS, pass1, 0)

        def pass2(g, d_acc):
            gs = pl.ds(g * 16, 16)
            ph = (idx_h[c, gs] & 1) * 64
            pr = (idx_r[c, gs] & 1) * 64
            pt = (idx_t[c, gs] & 1) * 64
            pn = (idx_n[c, gs] & 1) * 64
            ihv = inv_h[gs]
            itv = inv_t[gs]
            iqv = inv_n[gs]
            sp_v, s1_v, s2_v, sd_v = zero, zero, zero, zero
            for i in range(16):
                row = g * 16 + i
                ih = ihv[i]
                it = itv[i]
                iq = iqv[i]
                bh, br, bt, bn = ph[i], pr[i], pt[i], pn[i]
                acc_p = acc_1 = acc_2 = acc_d = None
                for k in range(4):
                    o = k * 16
                    hk = h_buf[row, pl.ds(bh + o, 16)]
                    rk = r_buf[row, pl.ds(br + o, 16)]
                    tk = t_buf[row, pl.ds(bt + o, 16)]
                    nk = n_buf[row, pl.ds(bn + o, 16)]
                    hn = hk * ih
                    tn = tk * it
                    nn = nk * iq
                    cc = hn + rk
                    bb = rk - tn
                    pv = cc - tn
                    n1 = bb + nn
                    n2 = cc - nn
                    dv = hk - tk
                    if acc_p is None:
                        acc_p, acc_1 = pv * pv, n1 * n1
                        acc_2, acc_d = n2 * n2, dv * dv
                    else:
                        acc_p = acc_p + pv * pv
                        acc_1 = acc_1 + n1 * n1
                        acc_2 = acc_2 + n2 * n2
                        acc_d = acc_d + dv * dv
                sp_v = jnp.where(lane == i, jnp.sum(acc_p), sp_v)
                s1_v = jnp.where(lane == i, jnp.sum(acc_1), s1_v)
                s2_v = jnp.where(lane == i, jnp.sum(acc_2), s2_v)
                sd_v = jnp.where(lane == i, jnp.sum(acc_d), sd_v)
            pos_b[gs] = -_sqrt(sp_v)
            neg1_b[gs] = -_sqrt(s1_v)
            neg2_b[gs] = -_sqrt(s2_v)
            return d_acc + _sqrt(sd_v)

        dist_acc = lax.fori_loop(0, GROUPS, pass2, dist_acc)

        pltpu.sync_copy(pos_b, pos_out.at[pl.ds(cbase, CHUNK)])
        pltpu.sync_copy(pos_b, pos_out.at[pl.ds(B + cbase, CHUNK)])
        pltpu.sync_copy(neg1_b, neg_out.at[pl.ds(cbase, CHUNK)])
        pltpu.sync_copy(neg2_b, neg_out.at[pl.ds(B + cbase, CHUNK)])
        return dist_acc

    dist_acc = lax.fori_loop(0, NCHUNK, chunk_body, zero)
    dist_b[...] = dist_acc
    pltpu.sync_copy(dist_b, dist_out.at[wid])


@functools.partial(jax.jit, static_argnames=())
def _sc_call(batch_head, batch_rel, batch_tail, batch_negative, ent2, rel2):
    mesh = plsc.VectorSubcoreMesh(core_axis_name="c", subcore_axis_name="s",
                                  num_cores=NC, num_subcores=NS)
    f = pl.kernel(
        _sc_body,
        out_type=(
            jax.ShapeDtypeStruct((2 * B,), _F32),
            jax.ShapeDtypeStruct((2 * B,), _F32),
            jax.ShapeDtypeStruct((NW, 16), _F32),
        ),
        mesh=mesh,
        compiler_params=pltpu.CompilerParams(needs_layout_passes=False),
        scratch_types=[
            pltpu.VMEM((NCHUNK, CHUNK), jnp.int32),
            pltpu.VMEM((NCHUNK, CHUNK), jnp.int32),
            pltpu.VMEM((NCHUNK, CHUNK), jnp.int32),
            pltpu.VMEM((NCHUNK, CHUNK), jnp.int32),
            pltpu.VMEM((CHUNK,), jnp.int32),
            pltpu.VMEM((CHUNK,), jnp.int32),
            pltpu.VMEM((CHUNK,), jnp.int32),
            pltpu.VMEM((CHUNK,), jnp.int32),
            pltpu.VMEM((CHUNK, PDIM), _F32),
            pltpu.VMEM((CHUNK, PDIM), _F32),
            pltpu.VMEM((CHUNK, PDIM), _F32),
            pltpu.VMEM((CHUNK, PDIM), _F32),
            pltpu.VMEM((CHUNK,), _F32),
            pltpu.VMEM((CHUNK,), _F32),
            pltpu.VMEM((CHUNK,), _F32),
            pltpu.VMEM((CHUNK,), _F32),
            pltpu.VMEM((CHUNK,), _F32),
            pltpu.VMEM((CHUNK,), _F32),
            pltpu.VMEM((16,), _F32),
            pltpu.SemaphoreType.DMA,
        ],
    )
    return f(batch_head, batch_rel, batch_tail, batch_negative, ent2, rel2)


def kernel(batch_head, batch_rel, batch_tail, batch_negative, ent_emb, rel_emb):
    ent2, rel2 = _pack_call(ent_emb.T, rel_emb.T)
    pos, neg, dist_parts = _sc_call(batch_head, batch_rel, batch_tail,
                                    batch_negative, ent2, rel2)
    return pos, neg, jnp.sum(dist_parts)


# speedup vs baseline: 2.5675x; 2.5675x over previous
import functools

import jax
import jax.numpy as jnp
from jax import lax
from jax.experimental import pallas as pl
from jax.experimental.pallas import tpu as pltpu
from jax.experimental.pallas import tpu_sc as plsc

B = 16384
DIM = 64
NC = 2
NS = 16
NW = NC * NS
ROWS_PER_W = B // NW
CHUNK = 128
NCHUNK = ROWS_PER_W // CHUNK
GROUPS = CHUNK // 16

_F32 = jnp.float32
_MAGIC = 0x5F3759DF


def _rsqrt(x):
    i = plsc.bitcast(x, jnp.int32)
    y = plsc.bitcast(jnp.int32(_MAGIC) - (i >> 1), _F32)
    for _ in range(3):
        y = y * (_F32(1.5) - _F32(0.5) * x * y * y)
    return y


def _sqrt(x):
    return x * _rsqrt(jnp.maximum(x, _F32(1e-30)))


def _row_sumsq(ref, row):
    a = ref[row, pl.ds(0, 16)]
    b = ref[row, pl.ds(16, 16)]
    c = ref[row, pl.ds(32, 16)]
    d = ref[row, pl.ds(48, 16)]
    return jnp.sum(a * a + b * b + c * c + d * d)


def _sc_body(head_hbm, rel_hbm, tail_hbm, negv_hbm, ent_hbm, relemb_hbm,
             pos_out, neg_out, dist_out,
             idx_h, idx_r, idx_t, idx_n,
             h_buf, r_buf, t_buf, n_buf,
             inv_h, inv_t, inv_n,
             pos_b, neg1_b, neg2_b, dist_b, sem):
    cid = lax.axis_index("c")
    sid = lax.axis_index("s")
    wid = sid * NC + cid
    base = wid * ROWS_PER_W
    lane = lax.broadcasted_iota(jnp.int32, (16,), 0)
    zero = jnp.zeros((16,), _F32)

    def chunk_body(c, dist_acc):
        cbase = base + c * CHUNK
        pltpu.sync_copy(head_hbm.at[pl.ds(cbase, CHUNK)], idx_h.at[c])
        pltpu.sync_copy(rel_hbm.at[pl.ds(cbase, CHUNK)], idx_r.at[c])
        pltpu.sync_copy(tail_hbm.at[pl.ds(cbase, CHUNK)], idx_t.at[c])
        pltpu.sync_copy(negv_hbm.at[pl.ds(cbase, CHUNK)], idx_n.at[c])
        cp_h = pltpu.async_copy(ent_hbm.at[idx_h.at[c]], h_buf, sem)
        cp_r = pltpu.async_copy(relemb_hbm.at[idx_r.at[c]], r_buf, sem)
        cp_t = pltpu.async_copy(ent_hbm.at[idx_t.at[c]], t_buf, sem)
        cp_n = pltpu.async_copy(ent_hbm.at[idx_n.at[c]], n_buf, sem)
        cp_h.wait()
        cp_r.wait()
        cp_t.wait()
        cp_n.wait()

        def pass1(g, carry):
            sh_v, st_v, sn_v = zero, zero, zero
            for i in range(16):
                row = g * 16 + i
                sh_v = jnp.where(lane == i, _row_sumsq(h_buf, row), sh_v)
                st_v = jnp.where(lane == i, _row_sumsq(t_buf, row), st_v)
                sn_v = jnp.where(lane == i, _row_sumsq(n_buf, row), sn_v)
            inv_h[pl.ds(g * 16, 16)] = _rsqrt(jnp.maximum(sh_v, _F32(1e-24)))
            inv_t[pl.ds(g * 16, 16)] = _rsqrt(jnp.maximum(st_v, _F32(1e-24)))
            inv_n[pl.ds(g * 16, 16)] = _rsqrt(jnp.maximum(sn_v, _F32(1e-24)))
            return carry

        lax.fori_loop(0, GROUPS, pass1, 0)

        def pass2(g, d_acc):
            sp_v, s1_v, s2_v, sd_v = zero, zero, zero, zero
            ihv = inv_h[pl.ds(g * 16, 16)]
            itv = inv_t[pl.ds(g * 16, 16)]
            iqv = inv_n[pl.ds(g * 16, 16)]
            for i in range(16):
                row = g * 16 + i
                ih = ihv[i]
                it = itv[i]
                iq = iqv[i]
                acc_p = acc_1 = acc_2 = acc_d = None
                for k in range(4):
                    sl = pl.ds(k * 16, 16)
                    hk = h_buf[row, sl]
                    rk = r_buf[row, sl]
                    tk = t_buf[row, sl]
                    nk = n_buf[row, sl]
                    hn = hk * ih
                    tn = tk * it
                    nn = nk * iq
                    cc = hn + rk
                    bb = rk - tn
                    pv = cc - tn
                    n1 = bb + nn
                    n2 = cc - nn
                    dv = hk - tk
                    if acc_p is None:
                        acc_p, acc_1 = pv * pv, n1 * n1
                        acc_2, acc_d = n2 * n2, dv * dv
                    else:
                        acc_p = acc_p + pv * pv
                        acc_1 = acc_1 + n1 * n1
                        acc_2 = acc_2 + n2 * n2
                        acc_d = acc_d + dv * dv
                sp_v = jnp.where(lane == i, jnp.sum(acc_p), sp_v)
                s1_v = jnp.where(lane == i, jnp.sum(acc_1), s1_v)
                s2_v = jnp.where(lane == i, jnp.sum(acc_2), s2_v)
                sd_v = jnp.where(lane == i, jnp.sum(acc_d), sd_v)
            gs = pl.ds(g * 16, 16)
            pos_b[gs] = -_sqrt(sp_v)
            neg1_b[gs] = -_sqrt(s1_v)
            neg2_b[gs] = -_sqrt(s2_v)
            return d_acc + _sqrt(sd_v)

        dist_acc = lax.fori_loop(0, GROUPS, pass2, dist_acc)

        pltpu.sync_copy(pos_b, pos_out.at[pl.ds(cbase, CHUNK)])
        pltpu.sync_copy(pos_b, pos_out.at[pl.ds(B + cbase, CHUNK)])
        pltpu.sync_copy(neg1_b, neg_out.at[pl.ds(cbase, CHUNK)])
        pltpu.sync_copy(neg2_b, neg_out.at[pl.ds(B + cbase, CHUNK)])
        return dist_acc

    dist_acc = lax.fori_loop(0, NCHUNK, chunk_body, zero)
    dist_b[...] = dist_acc
    pltpu.sync_copy(dist_b, dist_out.at[wid])


@functools.partial(jax.jit, static_argnames=())
def _sc_call(batch_head, batch_rel, batch_tail, batch_negative, ent_emb, rel_emb):
    mesh = plsc.VectorSubcoreMesh(core_axis_name="c", subcore_axis_name="s",
                                  num_cores=NC, num_subcores=NS)
    f = pl.kernel(
        _sc_body,
        out_type=(
            jax.ShapeDtypeStruct((2 * B,), _F32),
            jax.ShapeDtypeStruct((2 * B,), _F32),
            jax.ShapeDtypeStruct((NW, 16), _F32),
        ),
        mesh=mesh,
        compiler_params=pltpu.CompilerParams(needs_layout_passes=False,
                                             use_tc_tiling_on_sc=False),
        scratch_types=[
            pltpu.VMEM((NCHUNK, CHUNK), jnp.int32),
            pltpu.VMEM((NCHUNK, CHUNK), jnp.int32),
            pltpu.VMEM((NCHUNK, CHUNK), jnp.int32),
            pltpu.VMEM((NCHUNK, CHUNK), jnp.int32),
            pltpu.VMEM((CHUNK, DIM), _F32),
            pltpu.VMEM((CHUNK, DIM), _F32),
            pltpu.VMEM((CHUNK, DIM), _F32),
            pltpu.VMEM((CHUNK, DIM), _F32),
            pltpu.VMEM((CHUNK,), _F32),
            pltpu.VMEM((CHUNK,), _F32),
            pltpu.VMEM((CHUNK,), _F32),
            pltpu.VMEM((CHUNK,), _F32),
            pltpu.VMEM((CHUNK,), _F32),
            pltpu.VMEM((CHUNK,), _F32),
            pltpu.VMEM((16,), _F32),
            pltpu.SemaphoreType.DMA,
        ],
    )
    return f(batch_head, batch_rel, batch_tail, batch_negative, ent_emb, rel_emb)


def kernel(batch_head, batch_rel, batch_tail, batch_negative, ent_emb, rel_emb):
    pos, neg, dist_parts = _sc_call(batch_head, batch_rel, batch_tail,
                                    batch_negative, ent_emb, rel_emb)
    return pos, neg, jnp.sum(dist_parts)
